# baseline (speedup 1.0000x reference)
import functools

import jax
import jax.numpy as jnp
from jax import lax
from jax.experimental import pallas as pl
from jax.experimental.pallas import tpu as pltpu
from jax.experimental.pallas import tpu_sc as plsc

CH = 40


def _make_aggregate(N, E, D, Np):
    Dh = D // 2
    n_sub = 16
    epw = E // n_sub
    nch = epw // CH
    rpt = Np // n_sub
    NB = 8
    P = NB // 2
    assert epw % CH == 0 and Np % (8 * n_sub) == 0 and nch >= 2 * NB
    n_main = (nch - 2 * P) // NB

    mesh = plsc.VectorSubcoreMesh(core_axis_name="c", subcore_axis_name="s")

    @functools.partial(
        pl.kernel,
        out_type=jax.ShapeDtypeStruct((Np, D), jnp.float32),
        mesh=mesh,
        scratch_types=[
            [pltpu.VMEM((CH,), jnp.int32)] * NB,
            [pltpu.VMEM((CH, Dh), jnp.float32)] * NB,
            pltpu.VMEM_SHARED((Np, Dh), jnp.float32),
            [pltpu.SemaphoreType.DMA] * NB,
            [pltpu.SemaphoreType.DMA] * NB,
            [pltpu.SemaphoreType.DMA] * NB,
        ],
    )
    def agg(ea_hbm, ei_hbm, zeros_hbm, out_hbm, db, rb, acc, dsem, gsem, ssem):
        c = lax.axis_index("c")
        s = lax.axis_index("s")
        e_base = s * epw
        col = c * Dh

        pltpu.sync_copy(zeros_hbm, acc.at[pl.ds(s * rpt, rpt), :])
        plsc.subcore_barrier()

        def dst_src(k):
            return ei_hbm.at[pl.ds(E + e_base + k * CH, CH)]

        def rows_src(k):
            return ea_hbm.at[pl.ds(e_base + k * CH, CH), pl.ds(col, Dh)]

        def issue_loads(k, kb):
            pltpu.async_copy(dst_src(k), db[kb], dsem[kb])
            pltpu.async_copy(rows_src(k), rb[kb], gsem[kb])

        def step(k, kb, swait=True, prefetch=True):
            pltpu.make_async_copy(dst_src(k), db[kb], dsem[kb]).wait()
            pltpu.make_async_copy(rows_src(k), rb[kb], gsem[kb]).wait()
            if swait:
                kb2 = (kb - P) % NB
                pltpu.make_async_copy(rb[kb2], acc.at[db[kb2]], ssem[kb2]).wait()
            pltpu.async_copy(rb[kb], acc.at[db[kb]], ssem[kb], add=True)
            if prefetch:
                issue_loads(k + P, (kb + P) % NB)

        for k in range(P):
            issue_loads(k, k)
        for k in range(P):
            step(k, k, swait=False)

        def body(j, carry):
            for i in range(NB):
                step(NB * j + P + i, (P + i) % NB)
            return carry

        lax.fori_loop(0, n_main, body, 0)
        for k in range(NB * n_main + P, nch):
            step(k, k % NB, prefetch=(k + P < nch))
        for k in range(nch - (NB - P), nch):
            pltpu.make_async_copy(rb[k % NB], acc.at[db[k % NB]], ssem[k % NB]).wait()
        plsc.subcore_barrier()

        pltpu.sync_copy(
            acc.at[pl.ds(s * rpt, rpt), :],
            out_hbm.at[pl.ds(s * rpt, rpt), pl.ds(col, Dh)],
        )

    return agg


def _dot3(x, wh, wl):
    xh = x.astype(jnp.bfloat16)
    xl = (x - xh.astype(jnp.float32)).astype(jnp.bfloat16)
    acc = jnp.dot(xh, wl, preferred_element_type=jnp.float32)
    acc = acc + jnp.dot(xl, wh, preferred_element_type=jnp.float32)
    acc = acc + jnp.dot(xh, wh, preferred_element_type=jnp.float32)
    return acc


def _mlp1_body(nf, w1ah, w1al, b1, out):
    out[...] = _dot3(nf[...], w1ah[...], w1al[...]) + b1[...]


def _mlp2_body(hp, ag, nf, w1bh, w1bl, w2h, w2l, g, b, b2, out):
    h = hp[...] + _dot3(ag[...], w1bh[...], w1bl[...])
    mu = jnp.mean(h, axis=-1, keepdims=True)
    var = jnp.mean((h - mu) ** 2, axis=-1, keepdims=True)
    hn = (h - mu) * lax.rsqrt(var + 1e-5) * g[...] + b[...]
    hs = hn * jax.nn.sigmoid(hn)
    out[...] = nf[...] + _dot3(hs, w2h[...], w2l[...]) + b2[...]


def kernel(node_feat, edge_attr, edge_index, num_nodes, W1, b1, ln_g, ln_b, W2, b2):
    N, D = node_feat.shape
    E = edge_attr.shape[0]
    IN, H = W1.shape
    Dh = D // 2

    Np = ((N + 127) // 128) * 128
    ei_flat = edge_index.reshape(2 * E)
    zeros = jnp.zeros((Np // 16, Dh), jnp.float32)
    aggp = _make_aggregate(N, E, D, Np)(edge_attr, ei_flat, zeros)

    W1h = W1.astype(jnp.bfloat16)
    W1l = (W1 - W1h.astype(jnp.float32)).astype(jnp.bfloat16)
    W2h = W2.astype(jnp.bfloat16)
    W2l = (W2 - W2h.astype(jnp.float32)).astype(jnp.bfloat16)

    R = 1000
    grid = (N // R,)
    hpart = pl.pallas_call(
        _mlp1_body,
        grid=grid,
        in_specs=[
            pl.BlockSpec((R, D), lambda i: (i, 0)),
            pl.BlockSpec((D, H), lambda i: (0, 0)),
            pl.BlockSpec((D, H), lambda i: (0, 0)),
            pl.BlockSpec((1, H), lambda i: (0, 0)),
        ],
        out_specs=pl.BlockSpec((R, H), lambda i: (i, 0)),
        out_shape=jax.ShapeDtypeStruct((N, H), jnp.float32),
    )(node_feat, W1h[:D], W1l[:D], b1.reshape(1, H))

    out = pl.pallas_call(
        _mlp2_body,
        grid=grid,
        in_specs=[
            pl.BlockSpec((R, H), lambda i: (i, 0)),
            pl.BlockSpec((R, D), lambda i: (i, 0)),
            pl.BlockSpec((R, D), lambda i: (i, 0)),
            pl.BlockSpec((D, H), lambda i: (0, 0)),
            pl.BlockSpec((D, H), lambda i: (0, 0)),
            pl.BlockSpec((H, D), lambda i: (0, 0)),
            pl.BlockSpec((H, D), lambda i: (0, 0)),
            pl.BlockSpec((1, H), lambda i: (0, 0)),
            pl.BlockSpec((1, H), lambda i: (0, 0)),
            pl.BlockSpec((1, D), lambda i: (0, 0)),
        ],
        out_specs=pl.BlockSpec((R, D), lambda i: (i, 0)),
        out_shape=jax.ShapeDtypeStruct((N, D), jnp.float32),
    )(hpart, aggp, node_feat, W1h[D:], W1l[D:], W2h, W2l,
      ln_g.reshape(1, H), ln_b.reshape(1, H), b2.reshape(1, D))
    return out


# speedup vs baseline: 1.0044x; 1.0044x over previous
import functools

import jax
import jax.numpy as jnp
from jax import lax
from jax.experimental import pallas as pl
from jax.experimental.pallas import tpu as pltpu
from jax.experimental.pallas import tpu_sc as plsc

CH = 40


def _make_aggregate(N, E, D, Np):
    Dh = D // 2
    n_sub = 16
    epw = E // n_sub
    nch = epw // CH
    rpt = Np // n_sub
    NB = 8
    P = NB // 2
    assert NB == 2 * P
    assert epw % CH == 0 and Np % (8 * n_sub) == 0 and nch >= 2 * NB
    n_main = (nch - 2 * P) // NB

    mesh = plsc.VectorSubcoreMesh(core_axis_name="c", subcore_axis_name="s")

    @functools.partial(
        pl.kernel,
        out_type=jax.ShapeDtypeStruct((Np, D), jnp.float32),
        mesh=mesh,
        scratch_types=[
            [pltpu.VMEM((CH,), jnp.int32)] * NB,
            [pltpu.VMEM((CH, Dh), jnp.float32)] * NB,
            pltpu.VMEM_SHARED((Np, Dh), jnp.float32),
            [pltpu.SemaphoreType.DMA] * NB,
            [pltpu.SemaphoreType.DMA] * NB,
            [pltpu.SemaphoreType.DMA] * NB,
        ],
    )
    def agg(ea_hbm, ei_hbm, zeros_hbm, out_hbm, db, rb, acc, dsem, gsem, ssem):
        c = lax.axis_index("c")
        s = lax.axis_index("s")
        e_base = s * epw
        col = c * Dh

        pltpu.sync_copy(zeros_hbm, acc.at[pl.ds(s * rpt, rpt), :])
        plsc.subcore_barrier()

        def dst_src(k):
            return ei_hbm.at[pl.ds(E + e_base + k * CH, CH)]

        def rows_src(k):
            return ea_hbm.at[pl.ds(e_base + k * CH, CH), pl.ds(col, Dh)]

        def issue_loads(k, kb):
            pltpu.async_copy(dst_src(k), db[kb], dsem[kb])
            pltpu.async_copy(rows_src(k), rb[kb], gsem[kb])

        def step(k, kb, swait=True, prefetch=True):
            pltpu.make_async_copy(dst_src(k), db[kb], dsem[kb]).wait()
            pltpu.make_async_copy(rows_src(k), rb[kb], gsem[kb]).wait()
            if swait:
                kb2 = (kb - P) % NB
                pltpu.make_async_copy(rb[kb2], acc.at[db[kb2]], ssem[kb2]).wait()
            pltpu.async_copy(rb[kb], acc.at[db[kb]], ssem[kb], add=True)
            if prefetch:
                issue_loads(k + P, (kb + P) % NB)

        for k in range(P):
            issue_loads(k, k)
        for k in range(P):
            step(k, k, swait=False)

        def body(j, carry):
            for i in range(NB):
                step(NB * j + P + i, (P + i) % NB)
            return carry

        lax.fori_loop(0, n_main, body, 0)
        for k in range(NB * n_main + P, nch):
            step(k, k % NB, prefetch=(k + P < nch))
        for k in range(nch - (NB - P), nch):
            pltpu.make_async_copy(rb[k % NB], acc.at[db[k % NB]], ssem[k % NB]).wait()
        plsc.subcore_barrier()

        pltpu.sync_copy(
            acc.at[pl.ds(s * rpt, rpt), :],
            out_hbm.at[pl.ds(s * rpt, rpt), pl.ds(col, Dh)],
        )

    return agg


def _dot3(x, wh, wl):
    xh = x.astype(jnp.bfloat16)
    xl = (x - xh.astype(jnp.float32)).astype(jnp.bfloat16)
    acc = jnp.dot(xh, wl, preferred_element_type=jnp.float32)
    acc = acc + jnp.dot(xl, wh, preferred_element_type=jnp.float32)
    acc = acc + jnp.dot(xh, wh, preferred_element_type=jnp.float32)
    return acc


def _mlp1_body(nf, w1ah, w1al, b1, out):
    out[...] = _dot3(nf[...], w1ah[...], w1al[...]) + b1[...]


def _mlp2_body(hp, ag, nf, w1bh, w1bl, w2h, w2l, g, b, b2, out):
    h = hp[...] + _dot3(ag[...], w1bh[...], w1bl[...])
    mu = jnp.mean(h, axis=-1, keepdims=True)
    var = jnp.mean((h - mu) ** 2, axis=-1, keepdims=True)
    hn = (h - mu) * lax.rsqrt(var + 1e-5) * g[...] + b[...]
    hs = hn * jax.nn.sigmoid(hn)
    out[...] = nf[...] + _dot3(hs, w2h[...], w2l[...]) + b2[...]


def kernel(node_feat, edge_attr, edge_index, num_nodes, W1, b1, ln_g, ln_b, W2, b2):
    N, D = node_feat.shape
    E = edge_attr.shape[0]
    IN, H = W1.shape
    Dh = D // 2

    Np = ((N + 127) // 128) * 128
    ei_flat = edge_index.reshape(2 * E)
    zeros = jnp.zeros((Np // 16, Dh), jnp.float32)
    aggp = _make_aggregate(N, E, D, Np)(edge_attr, ei_flat, zeros)

    W1h = W1.astype(jnp.bfloat16)
    W1l = (W1 - W1h.astype(jnp.float32)).astype(jnp.bfloat16)
    W2h = W2.astype(jnp.bfloat16)
    W2l = (W2 - W2h.astype(jnp.float32)).astype(jnp.bfloat16)

    R = 2000
    grid = (N // R,)
    hpart = pl.pallas_call(
        _mlp1_body,
        grid=grid,
        in_specs=[
            pl.BlockSpec((R, D), lambda i: (i, 0)),
            pl.BlockSpec((D, H), lambda i: (0, 0)),
            pl.BlockSpec((D, H), lambda i: (0, 0)),
            pl.BlockSpec((1, H), lambda i: (0, 0)),
        ],
        out_specs=pl.BlockSpec((R, H), lambda i: (i, 0)),
        out_shape=jax.ShapeDtypeStruct((N, H), jnp.float32),
    )(node_feat, W1h[:D], W1l[:D], b1.reshape(1, H))

    out = pl.pallas_call(
        _mlp2_body,
        grid=grid,
        in_specs=[
            pl.BlockSpec((R, H), lambda i: (i, 0)),
            pl.BlockSpec((R, D), lambda i: (i, 0)),
            pl.BlockSpec((R, D), lambda i: (i, 0)),
            pl.BlockSpec((D, H), lambda i: (0, 0)),
            pl.BlockSpec((D, H), lambda i: (0, 0)),
            pl.BlockSpec((H, D), lambda i: (0, 0)),
            pl.BlockSpec((H, D), lambda i: (0, 0)),
            pl.BlockSpec((1, H), lambda i: (0, 0)),
            pl.BlockSpec((1, H), lambda i: (0, 0)),
            pl.BlockSpec((1, D), lambda i: (0, 0)),
        ],
        out_specs=pl.BlockSpec((R, D), lambda i: (i, 0)),
        out_shape=jax.ShapeDtypeStruct((N, D), jnp.float32),
    )(hpart, aggp, node_feat, W1h[D:], W1l[D:], W2h, W2l,
      ln_g.reshape(1, H), ln_b.reshape(1, H), b2.reshape(1, D))
    return out
